# Initial kernel scaffold; baseline (speedup 1.0000x reference)
import jax
import jax.numpy as jnp
from jax.experimental import pallas as pl


def kernel(actions, node_features, edge_index, W0a, b0a, W0b, b0b, W1a, b1a, W1b, b1b, Wm, bm, Wo, bo):
    raise NotImplementedError("write your pallas kernel here")


# speedup vs baseline: 8.4062x; 8.4062x over previous
import functools

import jax
import jax.numpy as jnp
from jax import lax
from jax.experimental import pallas as pl
from jax.experimental.pallas import tpu as pltpu
from jax.experimental.pallas import tpu_sc as plsc

B, N, E = 4, 10000, 160000
NN = B * N
H = 2 * N
BE = B * E
NC, NS = 2, 16
NW = NC * NS
C = 128
ITERS = -(-BE // (NW * C))
EW = ITERS * C
BE_PAD = NW * EW
HA = H + 8
STRIPE = 1256
STRIPE_Z = HA - (NS - 1) * STRIPE
STRIPE_X = H - (NS - 1) * STRIPE


@functools.lru_cache(maxsize=None)
def _make_segsum(D):
    mesh = plsc.VectorSubcoreMesh(
        core_axis_name="c", subcore_axis_name="s",
        num_cores=NC, num_subcores=NS)

    @functools.partial(
        pl.kernel,
        mesh=mesh,
        compiler_params=pltpu.CompilerParams(use_tc_tiling_on_sc=False),
        out_type=jax.ShapeDtypeStruct((NC, H, D), jnp.float32),
        scratch_types=[
            pltpu.VMEM((C,), jnp.int32),
            pltpu.VMEM((C,), jnp.int32),
            pltpu.VMEM((C, D), jnp.float32),
            pltpu.VMEM_SHARED((HA, D), jnp.float32),
            pltpu.SemaphoreType.DMA,
        ],
    )
    def seg(table_hbm, src_hbm, dst_hbm, zrows_hbm, out_hbm,
            src_v, dst_v, rows_v, acc_sh, sem):
        c = lax.axis_index("c")
        s = lax.axis_index("s")
        wid = s * NC + c

        @pl.when(s < NS - 1)
        def _():
            pltpu.sync_copy(zrows_hbm, acc_sh.at[pl.ds(s * STRIPE, STRIPE)])

        @pl.when(s == NS - 1)
        def _():
            pltpu.sync_copy(zrows_hbm.at[pl.ds(0, STRIPE_Z)],
                            acc_sh.at[pl.ds((NS - 1) * STRIPE, STRIPE_Z)])

        plsc.subcore_barrier()

        def body(i, carry):
            base = wid * EW + i * C
            pltpu.sync_copy(src_hbm.at[pl.ds(base, C)], src_v)
            pltpu.sync_copy(dst_hbm.at[pl.ds(base, C)], dst_v)
            pltpu.async_copy(table_hbm.at[src_v], rows_v, sem).wait()
            pltpu.sync_copy(rows_v, acc_sh.at[dst_v], add=True)
            return carry

        lax.fori_loop(0, ITERS, body, 0)
        plsc.subcore_barrier()

        @pl.when(s < NS - 1)
        def _():
            pltpu.sync_copy(acc_sh.at[pl.ds(s * STRIPE, STRIPE)],
                            out_hbm.at[c, pl.ds(s * STRIPE, STRIPE)])

        @pl.when(s == NS - 1)
        def _():
            pltpu.sync_copy(acc_sh.at[pl.ds((NS - 1) * STRIPE, STRIPE_X)],
                            out_hbm.at[c, pl.ds((NS - 1) * STRIPE, STRIPE_X)])

    return seg


RT = 2000
GT = NN // RT


def _mlp_body(x_ref, acc_ref, wa_ref, ba_ref, wb_ref, bb_ref, o_ref):
    i = pl.program_id(0)
    m = jnp.where(i >= GT // 2, 1.0, 0.0).astype(jnp.float32)
    xa = x_ref[...] + m * (acc_ref[0] + acc_ref[1])
    t = jnp.maximum(jnp.dot(xa, wa_ref[...],
                            preferred_element_type=jnp.float32) + ba_ref[...], 0.0)
    u = jnp.dot(t, wb_ref[...], preferred_element_type=jnp.float32) + bb_ref[...]
    o_ref[...] = jnp.maximum(u, 0.0)


def _mlp1(x0p, acc1, wa, ba, wb, bb):
    return pl.pallas_call(
        _mlp_body,
        grid=(GT,),
        in_specs=[
            pl.BlockSpec((RT, 16), lambda i: (i, 0)),
            pl.BlockSpec((NC, RT, 16), lambda i: (0, jnp.maximum(i - GT // 2, 0), 0)),
            pl.BlockSpec((16, 64), lambda i: (0, 0)),
            pl.BlockSpec((1, 64), lambda i: (0, 0)),
            pl.BlockSpec((64, 64), lambda i: (0, 0)),
            pl.BlockSpec((1, 64), lambda i: (0, 0)),
        ],
        out_specs=pl.BlockSpec((RT, 64), lambda i: (i, 0)),
        out_shape=jax.ShapeDtypeStruct((NN, 64), jnp.float32),
    )(x0p, acc1, wa, ba, wb, bb)


def _head_body(x_ref, acc_ref, wa_ref, ba_ref, wb_ref, bb_ref,
               wm_ref, bm_ref, wo_ref, bo_ref, o_ref, pool_ref):
    i = pl.program_id(0)
    m = jnp.where(i >= GT // 2, 1.0, 0.0).astype(jnp.float32)
    xa = x_ref[...] + m * (acc_ref[0] + acc_ref[1])
    t = jnp.maximum(jnp.dot(xa, wa_ref[...],
                            preferred_element_type=jnp.float32) + ba_ref[...], 0.0)
    u = jnp.dot(t, wb_ref[...], preferred_element_type=jnp.float32) + bb_ref[...]
    h2 = jnp.maximum(u, 0.0)

    @pl.when(i == 0)
    def _():
        pool_ref[...] = jnp.zeros_like(pool_ref)

    b = i // (GT // B)
    onehot = (lax.broadcasted_iota(jnp.int32, (B, 1), 0) == b).astype(jnp.float32)
    pool_ref[...] += onehot * jnp.sum(h2, axis=0, keepdims=True)

    @pl.when(i == GT - 1)
    def _():
        g = jnp.maximum(jnp.dot(pool_ref[...], wm_ref[...],
                                preferred_element_type=jnp.float32) + bm_ref[...], 0.0)
        z = jnp.dot(g, wo_ref[...], preferred_element_type=jnp.float32) + bo_ref[...]
        o_ref[...] = 1.0 / (1.0 + jnp.exp(-z))


def _mlp2_pool_head(h1, acc2, wa, ba, wb, bb, wm, bm, wo, bo):
    return pl.pallas_call(
        _head_body,
        grid=(GT,),
        in_specs=[
            pl.BlockSpec((RT, 64), lambda i: (i, 0)),
            pl.BlockSpec((NC, RT, 64), lambda i: (0, jnp.maximum(i - GT // 2, 0), 0)),
            pl.BlockSpec((64, 64), lambda i: (0, 0)),
            pl.BlockSpec((1, 64), lambda i: (0, 0)),
            pl.BlockSpec((64, 64), lambda i: (0, 0)),
            pl.BlockSpec((1, 64), lambda i: (0, 0)),
            pl.BlockSpec((64, 64), lambda i: (0, 0)),
            pl.BlockSpec((1, 64), lambda i: (0, 0)),
            pl.BlockSpec((64, N), lambda i: (0, 0)),
            pl.BlockSpec((1, N), lambda i: (0, 0)),
        ],
        out_specs=pl.BlockSpec((B, N), lambda i: (0, 0)),
        out_shape=jax.ShapeDtypeStruct((B, N), jnp.float32),
        scratch_shapes=[pltpu.VMEM((B, 64), jnp.float32)],
    )(h1, acc2, wa, ba, wb, bb, wm, bm, wo, bo)


def kernel(actions, node_features, edge_index, W0a, b0a, W0b, b0b,
           W1a, b1a, W1b, b1b, Wm, bm, Wo, bo):
    nf = node_features.reshape(B, N).astype(jnp.float32)
    x0 = jnp.stack((actions[:, :, 0], actions[:, :, 1], nf), axis=2).reshape(NN, 3)
    x0p = jnp.pad(x0, ((0, 0), (0, 13)))

    offs = (jnp.arange(B, dtype=edge_index.dtype) * N)[:, None, None]
    ei = (edge_index + offs).reshape(2, BE)
    src = jnp.concatenate(
        [ei[0], jnp.zeros((BE_PAD - BE,), dtype=jnp.int32)])
    dst = jnp.concatenate(
        [ei[1] - H, jnp.full((BE_PAD - BE,), H, dtype=jnp.int32)])

    z16 = jnp.zeros((STRIPE, 16), jnp.float32)
    z64 = jnp.zeros((STRIPE, 64), jnp.float32)

    acc1 = _make_segsum(16)(x0p[:H], src, dst, z16)
    h1 = _mlp1(x0p, acc1, jnp.pad(W0a, ((0, 13), (0, 0))),
               b0a.reshape(1, 64), W0b, b0b.reshape(1, 64))
    acc2 = _make_segsum(64)(h1[:H], src, dst, z64)
    out = _mlp2_pool_head(h1, acc2, W1a, b1a.reshape(1, 64), W1b,
                          b1b.reshape(1, 64), Wm, bm.reshape(1, 64),
                          Wo, bo.reshape(1, N))
    return out


# speedup vs baseline: 9.5489x; 1.1359x over previous
import functools

import jax
import jax.numpy as jnp
from jax import lax
from jax.experimental import pallas as pl
from jax.experimental.pallas import tpu as pltpu
from jax.experimental.pallas import tpu_sc as plsc

B, N, E = 4, 10000, 160000
NN = B * N
H = 2 * N
BE = B * E
NC, NS = 2, 16
NW = NC * NS
C = 128
NBUF = 4
ITERS = -(-BE // (NW * C * NBUF)) * NBUF
EW = ITERS * C
BE_PAD = NW * EW
HA = H + 8
STRIPE = 1256
STRIPE_Z = HA - (NS - 1) * STRIPE
STRIPE_X = H - (NS - 1) * STRIPE


@functools.lru_cache(maxsize=None)
def _make_segsum(D):
    mesh = plsc.VectorSubcoreMesh(
        core_axis_name="c", subcore_axis_name="s",
        num_cores=NC, num_subcores=NS)

    @functools.partial(
        pl.kernel,
        mesh=mesh,
        compiler_params=pltpu.CompilerParams(use_tc_tiling_on_sc=False),
        out_type=jax.ShapeDtypeStruct((NC, H, D), jnp.float32),
        scratch_types=[
            pltpu.VMEM((2, NBUF, C), jnp.int32),
            pltpu.VMEM((2, NBUF, C), jnp.int32),
            pltpu.VMEM((NBUF, C, D), jnp.float32),
            pltpu.VMEM_SHARED((HA, D), jnp.float32),
            pltpu.SemaphoreType.DMA,
        ] + [pltpu.SemaphoreType.DMA] * NBUF,
    )
    def seg(table_hbm, src_hbm, dst_hbm, zrows_hbm, out_hbm,
            src_v, dst_v, rows_v, acc_sh, isem, *gsems):
        c = lax.axis_index("c")
        s = lax.axis_index("s")
        wid = s * NC + c

        @pl.when(s < NS - 1)
        def _():
            pltpu.sync_copy(zrows_hbm, acc_sh.at[pl.ds(s * STRIPE, STRIPE)])

        @pl.when(s == NS - 1)
        def _():
            pltpu.sync_copy(zrows_hbm.at[pl.ds(0, STRIPE_Z)],
                            acc_sh.at[pl.ds((NS - 1) * STRIPE, STRIPE_Z)])

        plsc.subcore_barrier()

        NBLK = ITERS // NBUF

        def idx_copy(fn, p, sl):
            fn(src_hbm.at[wid, sl], src_v.at[p], isem)
            fn(dst_hbm.at[wid, sl], dst_v.at[p], isem)

        def gather(fn, p, b):
            return fn(table_hbm.at[src_v.at[p, b]], rows_v.at[b], gsems[b])

        pltpu.sync_copy(src_hbm.at[wid, pl.ds(0, NBUF)], src_v.at[0])
        pltpu.sync_copy(dst_hbm.at[wid, pl.ds(0, NBUF)], dst_v.at[0])
        for b in range(NBUF):
            gather(pltpu.async_copy, 0, b)
        idx_copy(pltpu.async_copy, 1, pl.ds(NBUF, NBUF))

        def body(t, carry):
            p = lax.rem(t, 2)
            q = lax.rem(t + 1, 2)
            sl = pl.ds((t + 1) * NBUF, NBUF)
            pltpu.make_async_copy(src_hbm.at[wid, sl], src_v.at[q], isem).wait()
            pltpu.make_async_copy(dst_hbm.at[wid, sl], dst_v.at[q], isem).wait()
            for b in range(NBUF):
                gather(pltpu.make_async_copy, p, b).wait()
                pltpu.sync_copy(rows_v.at[b], acc_sh.at[dst_v.at[p, b]],
                                add=True)
                gather(pltpu.async_copy, q, b)

            @pl.when(t + 2 < NBLK)
            def _():
                idx_copy(pltpu.async_copy, p, pl.ds((t + 2) * NBUF, NBUF))
            return carry

        lax.fori_loop(0, NBLK - 1, body, 0)
        pq = (NBLK - 1) % 2
        for b in range(NBUF):
            gather(pltpu.make_async_copy, pq, b).wait()
            pltpu.sync_copy(rows_v.at[b], acc_sh.at[dst_v.at[pq, b]], add=True)
        plsc.subcore_barrier()

        @pl.when(s < NS - 1)
        def _():
            pltpu.sync_copy(acc_sh.at[pl.ds(s * STRIPE, STRIPE)],
                            out_hbm.at[c, pl.ds(s * STRIPE, STRIPE)])

        @pl.when(s == NS - 1)
        def _():
            pltpu.sync_copy(acc_sh.at[pl.ds((NS - 1) * STRIPE, STRIPE_X)],
                            out_hbm.at[c, pl.ds((NS - 1) * STRIPE, STRIPE_X)])

    return seg


RT = 2000
GT = NN // RT


def _mlp_body(x_ref, acc_ref, wa_ref, ba_ref, wb_ref, bb_ref, o_ref):
    i = pl.program_id(0)
    m = jnp.where(i >= GT // 2, 1.0, 0.0).astype(jnp.float32)
    xa = x_ref[...] + m * (acc_ref[0] + acc_ref[1])
    t = jnp.maximum(jnp.dot(xa, wa_ref[...],
                            preferred_element_type=jnp.float32) + ba_ref[...], 0.0)
    u = jnp.dot(t, wb_ref[...], preferred_element_type=jnp.float32) + bb_ref[...]
    o_ref[...] = jnp.maximum(u, 0.0)


def _mlp1(x0p, acc1, wa, ba, wb, bb):
    return pl.pallas_call(
        _mlp_body,
        grid=(GT,),
        in_specs=[
            pl.BlockSpec((RT, 16), lambda i: (i, 0)),
            pl.BlockSpec((NC, RT, 16), lambda i: (0, jnp.maximum(i - GT // 2, 0), 0)),
            pl.BlockSpec((16, 64), lambda i: (0, 0)),
            pl.BlockSpec((1, 64), lambda i: (0, 0)),
            pl.BlockSpec((64, 64), lambda i: (0, 0)),
            pl.BlockSpec((1, 64), lambda i: (0, 0)),
        ],
        out_specs=pl.BlockSpec((RT, 64), lambda i: (i, 0)),
        out_shape=jax.ShapeDtypeStruct((NN, 64), jnp.float32),
    )(x0p, acc1, wa, ba, wb, bb)


def _head_body(x_ref, acc_ref, wa_ref, ba_ref, wb_ref, bb_ref,
               wm_ref, bm_ref, wo_ref, bo_ref, o_ref, pool_ref):
    i = pl.program_id(0)
    m = jnp.where(i >= GT // 2, 1.0, 0.0).astype(jnp.float32)
    xa = x_ref[...] + m * (acc_ref[0] + acc_ref[1])
    t = jnp.maximum(jnp.dot(xa, wa_ref[...],
                            preferred_element_type=jnp.float32) + ba_ref[...], 0.0)
    u = jnp.dot(t, wb_ref[...], preferred_element_type=jnp.float32) + bb_ref[...]
    h2 = jnp.maximum(u, 0.0)

    @pl.when(i == 0)
    def _():
        pool_ref[...] = jnp.zeros_like(pool_ref)

    b = i // (GT // B)
    onehot = (lax.broadcasted_iota(jnp.int32, (B, 1), 0) == b).astype(jnp.float32)
    pool_ref[...] += onehot * jnp.sum(h2, axis=0, keepdims=True)

    @pl.when(i == GT - 1)
    def _():
        g = jnp.maximum(jnp.dot(pool_ref[...], wm_ref[...],
                                preferred_element_type=jnp.float32) + bm_ref[...], 0.0)
        z = jnp.dot(g, wo_ref[...], preferred_element_type=jnp.float32) + bo_ref[...]
        o_ref[...] = 1.0 / (1.0 + jnp.exp(-z))


def _mlp2_pool_head(h1, acc2, wa, ba, wb, bb, wm, bm, wo, bo):
    return pl.pallas_call(
        _head_body,
        grid=(GT,),
        in_specs=[
            pl.BlockSpec((RT, 64), lambda i: (i, 0)),
            pl.BlockSpec((NC, RT, 64), lambda i: (0, jnp.maximum(i - GT // 2, 0), 0)),
            pl.BlockSpec((64, 64), lambda i: (0, 0)),
            pl.BlockSpec((1, 64), lambda i: (0, 0)),
            pl.BlockSpec((64, 64), lambda i: (0, 0)),
            pl.BlockSpec((1, 64), lambda i: (0, 0)),
            pl.BlockSpec((64, 64), lambda i: (0, 0)),
            pl.BlockSpec((1, 64), lambda i: (0, 0)),
            pl.BlockSpec((64, N), lambda i: (0, 0)),
            pl.BlockSpec((1, N), lambda i: (0, 0)),
        ],
        out_specs=pl.BlockSpec((B, N), lambda i: (0, 0)),
        out_shape=jax.ShapeDtypeStruct((B, N), jnp.float32),
        scratch_shapes=[pltpu.VMEM((B, 64), jnp.float32)],
    )(h1, acc2, wa, ba, wb, bb, wm, bm, wo, bo)


def kernel(actions, node_features, edge_index, W0a, b0a, W0b, b0b,
           W1a, b1a, W1b, b1b, Wm, bm, Wo, bo):
    nf = node_features.reshape(B, N).astype(jnp.float32)
    x0 = jnp.stack((actions[:, :, 0], actions[:, :, 1], nf), axis=2).reshape(NN, 3)
    x0p = jnp.pad(x0, ((0, 0), (0, 13)))

    offs = (jnp.arange(B, dtype=edge_index.dtype) * N)[:, None, None]
    ei = (edge_index + offs).reshape(2, BE)
    src = jnp.concatenate(
        [ei[0], jnp.zeros((BE_PAD - BE,), dtype=jnp.int32)]).reshape(
            NW, ITERS, C)
    dst = jnp.concatenate(
        [ei[1] - H, jnp.full((BE_PAD - BE,), H, dtype=jnp.int32)]).reshape(
            NW, ITERS, C)

    z16 = jnp.zeros((STRIPE, 16), jnp.float32)
    z64 = jnp.zeros((STRIPE, 64), jnp.float32)

    acc1 = _make_segsum(16)(x0p[:H], src, dst, z16)
    h1 = _mlp1(x0p, acc1, jnp.pad(W0a, ((0, 13), (0, 0))),
               b0a.reshape(1, 64), W0b, b0b.reshape(1, 64))
    acc2 = _make_segsum(64)(h1[:H], src, dst, z64)
    out = _mlp2_pool_head(h1, acc2, W1a, b1a.reshape(1, 64), W1b,
                          b1b.reshape(1, 64), Wm, bm.reshape(1, 64),
                          Wo, bo.reshape(1, N))
    return out


# speedup vs baseline: 23.3216x; 2.4423x over previous
import functools

import jax
import jax.numpy as jnp
from jax import lax
from jax.experimental import pallas as pl
from jax.experimental.pallas import tpu as pltpu
from jax.experimental.pallas import tpu_sc as plsc

B, N, E = 4, 10000, 160000
NN = B * N
H = 2 * N
BE = B * E
NC, NS = 2, 16
NW = NC * NS
C = 128
NBUF = 4
ITERS = -(-BE // (NW * C * NBUF)) * NBUF
EW = ITERS * C
BE_PAD = NW * EW
NDUM = 128
HA = H + NDUM
STRIPE = 1264
STRIPE_Z = HA - (NS - 1) * STRIPE
STRIPE_X = H - (NS - 1) * STRIPE


@functools.lru_cache(maxsize=None)
def _make_segsum(D):
    mesh = plsc.VectorSubcoreMesh(
        core_axis_name="c", subcore_axis_name="s",
        num_cores=NC, num_subcores=NS)

    @functools.partial(
        pl.kernel,
        mesh=mesh,
        compiler_params=pltpu.CompilerParams(use_tc_tiling_on_sc=False),
        out_type=jax.ShapeDtypeStruct((NC, H, D), jnp.float32),
        scratch_types=[
            pltpu.VMEM((2, NBUF, C), jnp.int32),
            pltpu.VMEM((2, NBUF, C), jnp.int32),
            pltpu.VMEM((NBUF, C, D), jnp.float32),
            pltpu.VMEM_SHARED((HA, D), jnp.float32),
            pltpu.SemaphoreType.DMA,
        ] + [pltpu.SemaphoreType.DMA] * NBUF,
    )
    def seg(table_hbm, src_hbm, dst_hbm, zrows_hbm, out_hbm,
            src_v, dst_v, rows_v, acc_sh, isem, *gsems):
        c = lax.axis_index("c")
        s = lax.axis_index("s")
        wid = s * NC + c

        @pl.when(s < NS - 1)
        def _():
            pltpu.sync_copy(zrows_hbm, acc_sh.at[pl.ds(s * STRIPE, STRIPE)])

        @pl.when(s == NS - 1)
        def _():
            pltpu.sync_copy(zrows_hbm.at[pl.ds(0, STRIPE_Z)],
                            acc_sh.at[pl.ds((NS - 1) * STRIPE, STRIPE_Z)])

        plsc.subcore_barrier()

        NBLK = ITERS // NBUF

        def idx_copy(fn, p, sl):
            fn(src_hbm.at[wid, sl], src_v.at[p], isem)
            fn(dst_hbm.at[wid, sl], dst_v.at[p], isem)

        def gather(fn, p, b):
            return fn(table_hbm.at[src_v.at[p, b]], rows_v.at[b], gsems[b])

        pltpu.sync_copy(src_hbm.at[wid, pl.ds(0, NBUF)], src_v.at[0])
        pltpu.sync_copy(dst_hbm.at[wid, pl.ds(0, NBUF)], dst_v.at[0])
        for b in range(NBUF):
            gather(pltpu.async_copy, 0, b)
        idx_copy(pltpu.async_copy, 1, pl.ds(NBUF, NBUF))

        def body(t, carry):
            p = lax.rem(t, 2)
            q = lax.rem(t + 1, 2)
            sl = pl.ds((t + 1) * NBUF, NBUF)
            pltpu.make_async_copy(src_hbm.at[wid, sl], src_v.at[q], isem).wait()
            pltpu.make_async_copy(dst_hbm.at[wid, sl], dst_v.at[q], isem).wait()
            for b in range(NBUF):
                gather(pltpu.make_async_copy, p, b).wait()
                pltpu.sync_copy(rows_v.at[b], acc_sh.at[dst_v.at[p, b]],
                                add=True)
                gather(pltpu.async_copy, q, b)

            @pl.when(t + 2 < NBLK)
            def _():
                idx_copy(pltpu.async_copy, p, pl.ds((t + 2) * NBUF, NBUF))
            return carry

        lax.fori_loop(0, NBLK - 1, body, 0)
        pq = (NBLK - 1) % 2
        for b in range(NBUF):
            gather(pltpu.make_async_copy, pq, b).wait()
            pltpu.sync_copy(rows_v.at[b], acc_sh.at[dst_v.at[pq, b]], add=True)
        plsc.subcore_barrier()

        @pl.when(s < NS - 1)
        def _():
            pltpu.sync_copy(acc_sh.at[pl.ds(s * STRIPE, STRIPE)],
                            out_hbm.at[c, pl.ds(s * STRIPE, STRIPE)])

        @pl.when(s == NS - 1)
        def _():
            pltpu.sync_copy(acc_sh.at[pl.ds((NS - 1) * STRIPE, STRIPE_X)],
                            out_hbm.at[c, pl.ds((NS - 1) * STRIPE, STRIPE_X)])

    return seg


RT = 2000
GT = NN // RT


GH = H // RT


def _mlp_a_body(x_ref, wa_ref, ba_ref, wb_ref, bb_ref, o_ref):
    t = jnp.maximum(jnp.dot(x_ref[...], wa_ref[...],
                            preferred_element_type=jnp.float32) + ba_ref[...], 0.0)
    u = jnp.dot(t, wb_ref[...], preferred_element_type=jnp.float32) + bb_ref[...]
    o_ref[...] = jnp.maximum(u, 0.0)


def _mlp_a(xh, wa, ba, wb, bb):
    return pl.pallas_call(
        _mlp_a_body,
        grid=(GH,),
        in_specs=[
            pl.BlockSpec((RT, 16), lambda i: (i, 0)),
            pl.BlockSpec((16, 64), lambda i: (0, 0)),
            pl.BlockSpec((1, 64), lambda i: (0, 0)),
            pl.BlockSpec((64, 64), lambda i: (0, 0)),
            pl.BlockSpec((1, 64), lambda i: (0, 0)),
        ],
        out_specs=pl.BlockSpec((RT, 64), lambda i: (i, 0)),
        out_shape=jax.ShapeDtypeStruct((H, 64), jnp.float32),
    )(xh, wa, ba, wb, bb)


def _mlp_b_body(x_ref, acc_ref, wa_ref, ba_ref, wb_ref, bb_ref, o_ref):
    xa = x_ref[...] + acc_ref[0] + acc_ref[1]
    t = jnp.maximum(jnp.dot(xa, wa_ref[...],
                            preferred_element_type=jnp.float32) + ba_ref[...], 0.0)
    u = jnp.dot(t, wb_ref[...], preferred_element_type=jnp.float32) + bb_ref[...]
    o_ref[...] = jnp.maximum(u, 0.0)


def _mlp_b(xh, acc1, wa, ba, wb, bb):
    return pl.pallas_call(
        _mlp_b_body,
        grid=(GH,),
        in_specs=[
            pl.BlockSpec((RT, 16), lambda i: (i, 0)),
            pl.BlockSpec((NC, RT, 16), lambda i: (0, i, 0)),
            pl.BlockSpec((16, 64), lambda i: (0, 0)),
            pl.BlockSpec((1, 64), lambda i: (0, 0)),
            pl.BlockSpec((64, 64), lambda i: (0, 0)),
            pl.BlockSpec((1, 64), lambda i: (0, 0)),
        ],
        out_specs=pl.BlockSpec((RT, 64), lambda i: (i, 0)),
        out_shape=jax.ShapeDtypeStruct((H, 64), jnp.float32),
    )(xh, acc1, wa, ba, wb, bb)


def _head_body(x_ref, acc_ref, wa_ref, ba_ref, wb_ref, bb_ref,
               wm_ref, bm_ref, wo_ref, bo_ref, o_ref, pool_ref):
    i = pl.program_id(0)
    m = jnp.where(i >= GT // 2, 1.0, 0.0).astype(jnp.float32)
    xa = x_ref[...] + m * (acc_ref[0] + acc_ref[1])
    t = jnp.maximum(jnp.dot(xa, wa_ref[...],
                            preferred_element_type=jnp.float32) + ba_ref[...], 0.0)
    u = jnp.dot(t, wb_ref[...], preferred_element_type=jnp.float32) + bb_ref[...]
    h2 = jnp.maximum(u, 0.0)

    @pl.when(i == 0)
    def _():
        pool_ref[...] = jnp.zeros_like(pool_ref)

    b = i // (GT // B)
    onehot = (lax.broadcasted_iota(jnp.int32, (B, 1), 0) == b).astype(jnp.float32)
    pool_ref[...] += onehot * jnp.sum(h2, axis=0, keepdims=True)

    @pl.when(i == GT - 1)
    def _():
        g = jnp.maximum(jnp.dot(pool_ref[...], wm_ref[...],
                                preferred_element_type=jnp.float32) + bm_ref[...], 0.0)
        z = jnp.dot(g, wo_ref[...], preferred_element_type=jnp.float32) + bo_ref[...]
        o_ref[...] = 1.0 / (1.0 + jnp.exp(-z))


def _mlp2_pool_head(h1, acc2, wa, ba, wb, bb, wm, bm, wo, bo):
    return pl.pallas_call(
        _head_body,
        grid=(GT,),
        in_specs=[
            pl.BlockSpec((RT, 64), lambda i: (i, 0)),
            pl.BlockSpec((NC, RT, 64), lambda i: (0, jnp.maximum(i - GT // 2, 0), 0)),
            pl.BlockSpec((64, 64), lambda i: (0, 0)),
            pl.BlockSpec((1, 64), lambda i: (0, 0)),
            pl.BlockSpec((64, 64), lambda i: (0, 0)),
            pl.BlockSpec((1, 64), lambda i: (0, 0)),
            pl.BlockSpec((64, 64), lambda i: (0, 0)),
            pl.BlockSpec((1, 64), lambda i: (0, 0)),
            pl.BlockSpec((64, N), lambda i: (0, 0)),
            pl.BlockSpec((1, N), lambda i: (0, 0)),
        ],
        out_specs=pl.BlockSpec((B, N), lambda i: (0, 0)),
        out_shape=jax.ShapeDtypeStruct((B, N), jnp.float32),
        scratch_shapes=[pltpu.VMEM((B, 64), jnp.float32)],
    )(h1, acc2, wa, ba, wb, bb, wm, bm, wo, bo)


def kernel(actions, node_features, edge_index, W0a, b0a, W0b, b0b,
           W1a, b1a, W1b, b1b, Wm, bm, Wo, bo):
    nf = node_features.reshape(B, N).astype(jnp.float32)
    x0 = jnp.stack((actions[:, :, 0], actions[:, :, 1], nf), axis=2).reshape(NN, 3)
    x0p = jnp.pad(x0, ((0, 0), (0, 13)))

    offs = (jnp.arange(B, dtype=edge_index.dtype) * N)[:, None, None]
    ei = (edge_index + offs).reshape(2, BE)
    spread = jnp.arange(BE_PAD - BE, dtype=jnp.int32) % NDUM
    src = jnp.concatenate([ei[0], spread]).reshape(NW, ITERS, C)
    dst = jnp.concatenate([ei[1] - H, H + spread]).reshape(NW, ITERS, C)

    z16 = jnp.zeros((STRIPE, 16), jnp.float32)
    z64 = jnp.zeros((STRIPE, 64), jnp.float32)

    W0a_p = jnp.pad(W0a, ((0, 13), (0, 0)))
    acc1 = _make_segsum(16)(x0p[:H], src, dst, z16)
    h1a = _mlp_a(x0p[:H], W0a_p, b0a.reshape(1, 64), W0b, b0b.reshape(1, 64))
    acc2 = _make_segsum(64)(h1a, src, dst, z64)
    h1b = _mlp_b(x0p[H:], acc1, W0a_p, b0a.reshape(1, 64), W0b,
                 b0b.reshape(1, 64))
    h1 = jnp.concatenate([h1a, h1b])
    out = _mlp2_pool_head(h1, acc2, W1a, b1a.reshape(1, 64), W1b,
                          b1b.reshape(1, 64), Wm, bm.reshape(1, 64),
                          Wo, bo.reshape(1, N))
    return out
